# baseline (speedup 1.0000x reference)
import jax
import jax.numpy as jnp
from jax.experimental import pallas as pl
from jax.experimental.pallas import tpu as pltpu

N_CODES = 1024
DIM = 256
ROWS = 8192
HW = 1024
BLOCK_ROWS = 1024
NUM_BLOCKS = ROWS // BLOCK_ROWS
BETA_C = 0.25
TOTAL_ELEMS = ROWS * DIM


def _vq_body(xt_ref, cb_ref, a_ref, b_ref,
             zq_ref, codes_ref, idx_ref, loss_ref, perp_ref,
             hist_ref, sqs_ref):
    i = pl.program_id(0)

    @pl.when(i == 0)
    def _init():
        hist_ref[:] = jnp.zeros_like(hist_ref)
        sqs_ref[0] = 0.0

    x = jnp.transpose(xt_ref[:])
    cb = cb_ref[:]
    xc = jax.lax.dot_general(
        x, cb, (((1,), (1,)), ((), ())),
        preferred_element_type=jnp.float32)
    dist = (a_ref[:] + b_ref[:]) - 2.0 * xc
    dmin = jnp.min(dist, axis=1, keepdims=True)
    lane = jax.lax.broadcasted_iota(jnp.int32, dist.shape, 1)
    idx = jnp.min(jnp.where(dist == dmin, lane, N_CODES), axis=1
                  ).astype(jnp.int32)
    codes = (lane == idx[:, None]).astype(jnp.float32)
    codes_ref[:] = codes
    zq = jax.lax.dot_general(
        codes, cb, (((1,), (0,)), ((), ())),
        preferred_element_type=jnp.float32)
    zq_ref[:] = jnp.transpose(zq)
    idx_ref[:] = idx[:, None]
    diff = x - zq
    sqs_ref[0] += jnp.sum(diff * diff)
    ones_row = jnp.ones((1, BLOCK_ROWS), dtype=jnp.float32)
    hist_ref[:] += jax.lax.dot_general(
        ones_row, codes, (((1,), (0,)), ((), ())),
        preferred_element_type=jnp.float32)

    @pl.when(i == NUM_BLOCKS - 1)
    def _fin():
        loss_ref[:] = jnp.full(
            (1, 1), sqs_ref[0] * ((1.0 + BETA_C) / TOTAL_ELEMS),
            dtype=jnp.float32)
        e_mean = hist_ref[:] * (1.0 / ROWS)
        perp_ref[:] = jnp.exp(
            -jnp.sum(e_mean * jnp.log(e_mean + 1e-10),
                     axis=1, keepdims=True))


def kernel(z, codebook):
    bsz, ch, h, w = z.shape
    z2 = z.reshape(bsz * ch, HW)
    row_nrm = jnp.sum(z * z, axis=1).reshape(ROWS, 1)
    code_nrm = jnp.sum(codebook ** 2, axis=1)[None, :]

    zq2, codes, idx, loss, perp = pl.pallas_call(
        _vq_body,
        grid=(NUM_BLOCKS,),
        in_specs=[
            pl.BlockSpec((DIM, HW), lambda i: (i, 0)),
            pl.BlockSpec((N_CODES, DIM), lambda i: (0, 0)),
            pl.BlockSpec((BLOCK_ROWS, 1), lambda i: (i, 0)),
            pl.BlockSpec((1, N_CODES), lambda i: (0, 0)),
        ],
        out_specs=[
            pl.BlockSpec((DIM, HW), lambda i: (i, 0)),
            pl.BlockSpec((BLOCK_ROWS, N_CODES), lambda i: (i, 0)),
            pl.BlockSpec((BLOCK_ROWS, 1), lambda i: (i, 0)),
            pl.BlockSpec((1, 1), lambda i: (0, 0)),
            pl.BlockSpec((1, 1), lambda i: (0, 0)),
        ],
        out_shape=[
            jax.ShapeDtypeStruct((bsz * ch, HW), jnp.float32),
            jax.ShapeDtypeStruct((ROWS, N_CODES), jnp.float32),
            jax.ShapeDtypeStruct((ROWS, 1), jnp.int32),
            jax.ShapeDtypeStruct((1, 1), jnp.float32),
            jax.ShapeDtypeStruct((1, 1), jnp.float32),
        ],
        scratch_shapes=[
            pltpu.VMEM((1, N_CODES), jnp.float32),
            pltpu.SMEM((1,), jnp.float32),
        ],
        compiler_params=pltpu.CompilerParams(
            dimension_semantics=("arbitrary",)),
    )(z2, codebook, row_nrm, code_nrm)

    z_q_out = zq2.reshape(bsz, ch, h, w)
    return (loss[0, 0], z_q_out, codes, idx, perp[0, 0])


# speedup vs baseline: 1.2502x; 1.2502x over previous
import jax
import jax.numpy as jnp
from jax.experimental import pallas as pl
from jax.experimental.pallas import tpu as pltpu

N_CODES = 1024
DIM = 256
ROWS = 8192
BLOCK_ROWS = 1024
NUM_BLOCKS = ROWS // BLOCK_ROWS
BETA_C = 0.25
TOTAL_ELEMS = ROWS * DIM


def _vq_body(x_ref, cb_ref, a_ref, b_ref,
             codes_ref, idx_ref, loss_ref, perp_ref,
             hist_ref, sqs_ref):
    i = pl.program_id(0)

    @pl.when(i == 0)
    def _init():
        hist_ref[:] = jnp.zeros_like(hist_ref)
        sqs_ref[0] = 0.0

    x = x_ref[:]
    cb = cb_ref[:]
    xc = jax.lax.dot_general(
        x, cb, (((1,), (1,)), ((), ())),
        preferred_element_type=jnp.float32)
    dist = (a_ref[:] + b_ref[:]) - 2.0 * xc
    dmin = jnp.min(dist, axis=1, keepdims=True)
    lane = jax.lax.broadcasted_iota(jnp.int32, dist.shape, 1)
    idx = jnp.min(jnp.where(dist == dmin, lane, N_CODES), axis=1
                  ).astype(jnp.int32)
    codes = (lane == idx[:, None]).astype(jnp.float32)
    codes_ref[:] = codes
    zq = jax.lax.dot_general(
        codes, cb, (((1,), (0,)), ((), ())),
        preferred_element_type=jnp.float32)
    idx_ref[:] = idx[:, None]
    diff = x - zq
    sqs_ref[0] += jnp.sum(diff * diff)
    ones_row = jnp.ones((1, BLOCK_ROWS), dtype=jnp.float32)
    hist_ref[:] += jax.lax.dot_general(
        ones_row, codes, (((1,), (0,)), ((), ())),
        preferred_element_type=jnp.float32)

    @pl.when(i == NUM_BLOCKS - 1)
    def _fin():
        loss_ref[:] = jnp.full(
            (1, 1), sqs_ref[0] * ((1.0 + BETA_C) / TOTAL_ELEMS),
            dtype=jnp.float32)
        e_mean = hist_ref[:] * (1.0 / ROWS)
        perp_ref[:] = jnp.exp(
            -jnp.sum(e_mean * jnp.log(e_mean + 1e-10),
                     axis=1, keepdims=True))


def kernel(z, codebook):
    bsz, ch, h, w = z.shape
    zp_flat = jnp.transpose(z, (0, 2, 3, 1)).reshape(ROWS, DIM)
    row_nrm = jnp.sum(zp_flat ** 2, axis=1, keepdims=True)
    code_nrm = jnp.sum(codebook ** 2, axis=1)[None, :]

    codes, idx, loss, perp = pl.pallas_call(
        _vq_body,
        grid=(NUM_BLOCKS,),
        in_specs=[
            pl.BlockSpec((BLOCK_ROWS, DIM), lambda i: (i, 0)),
            pl.BlockSpec((N_CODES, DIM), lambda i: (0, 0)),
            pl.BlockSpec((BLOCK_ROWS, 1), lambda i: (i, 0)),
            pl.BlockSpec((1, N_CODES), lambda i: (0, 0)),
        ],
        out_specs=[
            pl.BlockSpec((BLOCK_ROWS, N_CODES), lambda i: (i, 0)),
            pl.BlockSpec((BLOCK_ROWS, 1), lambda i: (i, 0)),
            pl.BlockSpec((1, 1), lambda i: (0, 0)),
            pl.BlockSpec((1, 1), lambda i: (0, 0)),
        ],
        out_shape=[
            jax.ShapeDtypeStruct((ROWS, N_CODES), jnp.float32),
            jax.ShapeDtypeStruct((ROWS, 1), jnp.int32),
            jax.ShapeDtypeStruct((1, 1), jnp.float32),
            jax.ShapeDtypeStruct((1, 1), jnp.float32),
        ],
        scratch_shapes=[
            pltpu.VMEM((1, N_CODES), jnp.float32),
            pltpu.SMEM((1,), jnp.float32),
        ],
        compiler_params=pltpu.CompilerParams(
            dimension_semantics=("arbitrary",)),
    )(zp_flat, codebook, row_nrm, code_nrm)

    z_q_out = jnp.transpose(
        jnp.take(codebook, idx[:, 0], axis=0).reshape(bsz, h, w, ch),
        (0, 3, 1, 2))
    return (loss[0, 0], z_q_out, codes, idx, perp[0, 0])


# speedup vs baseline: 2.3180x; 1.8541x over previous
import jax
import jax.numpy as jnp
from jax.experimental import pallas as pl
from jax.experimental.pallas import tpu as pltpu

N_CODES = 1024
DIM = 256
ROWS = 8192
BLOCK_ROWS = 1024
NUM_BLOCKS = ROWS // BLOCK_ROWS
BETA_C = 0.25
TOTAL_ELEMS = ROWS * DIM


def _vq_body(x_ref, cb_ref, a_ref, b_ref,
             zq_ref, codes_ref, idx_ref, hist_ref, sqs_ref):
    x = x_ref[:]
    cb = cb_ref[:]
    xc = jax.lax.dot_general(
        x, cb, (((1,), (1,)), ((), ())),
        preferred_element_type=jnp.float32)
    dist = (a_ref[:] + b_ref[:]) - 2.0 * xc
    dmin = jnp.min(dist, axis=1, keepdims=True)
    lane = jax.lax.broadcasted_iota(jnp.int32, dist.shape, 1)
    idx = jnp.min(jnp.where(dist == dmin, lane, N_CODES), axis=1
                  ).astype(jnp.int32)
    codes = (lane == idx[:, None]).astype(jnp.float32)
    codes_ref[:] = codes
    zq = jax.lax.dot_general(
        codes, cb, (((1,), (0,)), ((), ())),
        preferred_element_type=jnp.float32)
    zq_ref[:] = zq
    idx_ref[:] = idx[:, None]
    diff = x - zq
    sqs_ref[:] = jnp.full((1, 1, 128), jnp.sum(diff * diff),
                          dtype=jnp.float32)
    ones_row = jnp.ones((1, BLOCK_ROWS), dtype=jnp.float32)
    hist_ref[:] = jax.lax.dot_general(
        ones_row, codes, (((1,), (0,)), ((), ())),
        preferred_element_type=jnp.float32)[None]


def _fin_body(histp_ref, sqsp_ref, loss_ref, perp_ref):
    sqs = jnp.sum(sqsp_ref[:, :, 0])
    loss_ref[:] = jnp.full(
        (1, 1), sqs * ((1.0 + BETA_C) / TOTAL_ELEMS), dtype=jnp.float32)
    e_mean = jnp.sum(histp_ref[:, 0, :], axis=0, keepdims=True) * (1.0 / ROWS)
    perp_ref[:] = jnp.exp(
        -jnp.sum(e_mean * jnp.log(e_mean + 1e-10), axis=1, keepdims=True))


def kernel(z, codebook):
    bsz, ch, h, w = z.shape
    zp_flat = jnp.transpose(z, (0, 2, 3, 1)).reshape(ROWS, DIM)
    row_nrm = jnp.sum(zp_flat ** 2, axis=1, keepdims=True)
    code_nrm = jnp.sum(codebook ** 2, axis=1)[None, :]

    zq_flat, codes, idx, histp, sqsp = pl.pallas_call(
        _vq_body,
        grid=(NUM_BLOCKS,),
        in_specs=[
            pl.BlockSpec((BLOCK_ROWS, DIM), lambda i: (i, 0)),
            pl.BlockSpec((N_CODES, DIM), lambda i: (0, 0)),
            pl.BlockSpec((BLOCK_ROWS, 1), lambda i: (i, 0)),
            pl.BlockSpec((1, N_CODES), lambda i: (0, 0)),
        ],
        out_specs=[
            pl.BlockSpec((BLOCK_ROWS, DIM), lambda i: (i, 0)),
            pl.BlockSpec((BLOCK_ROWS, N_CODES), lambda i: (i, 0)),
            pl.BlockSpec((BLOCK_ROWS, 1), lambda i: (i, 0)),
            pl.BlockSpec((1, 1, N_CODES), lambda i: (i, 0, 0)),
            pl.BlockSpec((1, 1, 128), lambda i: (i, 0, 0)),
        ],
        out_shape=[
            jax.ShapeDtypeStruct((ROWS, DIM), jnp.float32),
            jax.ShapeDtypeStruct((ROWS, N_CODES), jnp.float32),
            jax.ShapeDtypeStruct((ROWS, 1), jnp.int32),
            jax.ShapeDtypeStruct((NUM_BLOCKS, 1, N_CODES), jnp.float32),
            jax.ShapeDtypeStruct((NUM_BLOCKS, 1, 128), jnp.float32),
        ],
        compiler_params=pltpu.CompilerParams(
            dimension_semantics=("parallel",)),
    )(zp_flat, codebook, row_nrm, code_nrm)

    loss, perp = pl.pallas_call(
        _fin_body,
        out_shape=[
            jax.ShapeDtypeStruct((1, 1), jnp.float32),
            jax.ShapeDtypeStruct((1, 1), jnp.float32),
        ],
    )(histp, sqsp)

    z_q_out = jnp.transpose(zq_flat.reshape(bsz, h, w, ch), (0, 3, 1, 2))
    return (loss[0, 0], z_q_out, codes, idx, perp[0, 0])


# speedup vs baseline: 2.3352x; 1.0074x over previous
import jax
import jax.numpy as jnp
from jax.experimental import pallas as pl
from jax.experimental.pallas import tpu as pltpu

N_CODES = 1024
DIM = 256
ROWS = 8192
BLOCK_ROWS = 1024
NUM_BLOCKS = ROWS // BLOCK_ROWS
BETA_C = 0.25
TOTAL_ELEMS = ROWS * DIM


def _vq_body(x_ref, cb_ref, a_ref, b_ref,
             zq_ref, codes_ref, idx_ref, loss_ref, perp_ref,
             hist_ref, sqs_ref):
    i = pl.program_id(0)

    @pl.when(i == 0)
    def _init():
        hist_ref[:] = jnp.zeros_like(hist_ref)
        sqs_ref[0] = 0.0

    x = x_ref[:]
    cb = cb_ref[:]
    xc = jax.lax.dot_general(
        x, cb, (((1,), (1,)), ((), ())),
        preferred_element_type=jnp.float32)
    dist = (a_ref[:] + b_ref[:]) - 2.0 * xc
    dmin = jnp.min(dist, axis=1, keepdims=True)
    lane = jax.lax.broadcasted_iota(jnp.int32, dist.shape, 1)
    idx = jnp.min(jnp.where(dist == dmin, lane, N_CODES), axis=1
                  ).astype(jnp.int32)
    codes = (lane == idx[:, None]).astype(jnp.float32)
    codes_ref[:] = codes
    zq = jax.lax.dot_general(
        codes, cb, (((1,), (0,)), ((), ())),
        preferred_element_type=jnp.float32)
    zq_ref[:] = zq
    idx_ref[:] = idx[:, None]
    diff = x - zq
    sqs_ref[0] += jnp.sum(diff * diff)
    ones_row = jnp.ones((1, BLOCK_ROWS), dtype=jnp.float32)
    hist_ref[:] += jax.lax.dot_general(
        ones_row, codes, (((1,), (0,)), ((), ())),
        preferred_element_type=jnp.float32)

    @pl.when(i == NUM_BLOCKS - 1)
    def _fin():
        loss_ref[:] = jnp.full(
            (1, 1), sqs_ref[0] * ((1.0 + BETA_C) / TOTAL_ELEMS),
            dtype=jnp.float32)
        e_mean = hist_ref[:] * (1.0 / ROWS)
        perp_ref[:] = jnp.exp(
            -jnp.sum(e_mean * jnp.log(e_mean + 1e-10),
                     axis=1, keepdims=True))


def kernel(z, codebook):
    bsz, ch, h, w = z.shape
    zp_flat = jnp.transpose(z, (0, 2, 3, 1)).reshape(ROWS, DIM)
    row_nrm = jnp.sum(z * z, axis=1).reshape(ROWS, 1)
    code_nrm = jnp.sum(codebook ** 2, axis=1)[None, :]

    zq_flat, codes, idx, loss, perp = pl.pallas_call(
        _vq_body,
        grid=(NUM_BLOCKS,),
        in_specs=[
            pl.BlockSpec((BLOCK_ROWS, DIM), lambda i: (i, 0)),
            pl.BlockSpec((N_CODES, DIM), lambda i: (0, 0)),
            pl.BlockSpec((BLOCK_ROWS, 1), lambda i: (i, 0)),
            pl.BlockSpec((1, N_CODES), lambda i: (0, 0)),
        ],
        out_specs=[
            pl.BlockSpec((BLOCK_ROWS, DIM), lambda i: (i, 0)),
            pl.BlockSpec((BLOCK_ROWS, N_CODES), lambda i: (i, 0)),
            pl.BlockSpec((BLOCK_ROWS, 1), lambda i: (i, 0)),
            pl.BlockSpec((1, 1), lambda i: (0, 0)),
            pl.BlockSpec((1, 1), lambda i: (0, 0)),
        ],
        out_shape=[
            jax.ShapeDtypeStruct((ROWS, DIM), jnp.float32),
            jax.ShapeDtypeStruct((ROWS, N_CODES), jnp.float32),
            jax.ShapeDtypeStruct((ROWS, 1), jnp.int32),
            jax.ShapeDtypeStruct((1, 1), jnp.float32),
            jax.ShapeDtypeStruct((1, 1), jnp.float32),
        ],
        scratch_shapes=[
            pltpu.VMEM((1, N_CODES), jnp.float32),
            pltpu.SMEM((1,), jnp.float32),
        ],
        compiler_params=pltpu.CompilerParams(
            dimension_semantics=("arbitrary",)),
    )(zp_flat, codebook, row_nrm, code_nrm)

    z_q_out = jnp.transpose(zq_flat.reshape(bsz, h, w, ch), (0, 3, 1, 2))
    return (loss[0, 0], z_q_out, codes, idx, perp[0, 0])


# speedup vs baseline: 2.3465x; 1.0049x over previous
import jax
import jax.numpy as jnp
from jax.experimental import pallas as pl
from jax.experimental.pallas import tpu as pltpu

N_CODES = 1024
DIM = 256
ROWS = 8192
BLOCK_ROWS = 1024
NUM_BLOCKS = ROWS // BLOCK_ROWS
BETA_C = 0.25
TOTAL_ELEMS = ROWS * DIM


def _vq_body(x_ref, cb_ref, a_ref, b_ref,
             zq_ref, codes_ref, idx_ref, loss_ref, perp_ref,
             hist_ref, sqs_ref):
    i = pl.program_id(0)

    @pl.when(i == 0)
    def _init():
        hist_ref[:] = jnp.zeros_like(hist_ref)
        sqs_ref[0] = 0.0

    x = x_ref[:]
    cb = cb_ref[:]
    xc = jax.lax.dot_general(
        x, cb, (((1,), (1,)), ((), ())),
        preferred_element_type=jnp.float32)
    dist = (a_ref[:] + b_ref[:]) - 2.0 * xc
    dmin = jnp.min(dist, axis=1, keepdims=True)
    lane = jax.lax.broadcasted_iota(jnp.int32, dist.shape, 1)
    idx = jnp.min(jnp.where(dist == dmin, lane, N_CODES), axis=1
                  ).astype(jnp.int32)
    codes = (lane == idx[:, None]).astype(jnp.float32)
    codes_ref[:] = codes
    zq = jax.lax.dot_general(
        codes, cb, (((1,), (0,)), ((), ())),
        preferred_element_type=jnp.float32)
    zq_ref[:] = zq
    idx_ref[:] = idx[:, None]
    diff = x - zq
    sqs_ref[0] += jnp.sum(diff * diff)
    ones_row = jnp.ones((1, BLOCK_ROWS), dtype=jnp.float32)
    hist_ref[:] += jax.lax.dot_general(
        ones_row, codes, (((1,), (0,)), ((), ())),
        preferred_element_type=jnp.float32)

    @pl.when(i == NUM_BLOCKS - 1)
    def _fin():
        loss_ref[:] = jnp.full(
            (1, 1), sqs_ref[0] * ((1.0 + BETA_C) / TOTAL_ELEMS),
            dtype=jnp.float32)
        e_mean = hist_ref[:] * (1.0 / ROWS)
        perp_ref[:] = jnp.exp(
            -jnp.sum(e_mean * jnp.log(e_mean + 1e-10),
                     axis=1, keepdims=True))


def kernel(z, codebook):
    bsz, ch, h, w = z.shape
    zp_flat = jnp.transpose(z, (0, 2, 3, 1)).reshape(ROWS, DIM)
    row_nrm = jnp.sum(zp_flat ** 2, axis=1, keepdims=True)
    code_nrm = jnp.sum(codebook ** 2, axis=1)[None, :]

    zq_flat, codes, idx, loss, perp = pl.pallas_call(
        _vq_body,
        grid=(NUM_BLOCKS,),
        in_specs=[
            pl.BlockSpec((BLOCK_ROWS, DIM), lambda i: (i, 0)),
            pl.BlockSpec((N_CODES, DIM), lambda i: (0, 0)),
            pl.BlockSpec((BLOCK_ROWS, 1), lambda i: (i, 0)),
            pl.BlockSpec((1, N_CODES), lambda i: (0, 0)),
        ],
        out_specs=[
            pl.BlockSpec((BLOCK_ROWS, DIM), lambda i: (i, 0)),
            pl.BlockSpec((BLOCK_ROWS, N_CODES), lambda i: (i, 0)),
            pl.BlockSpec((BLOCK_ROWS, 1), lambda i: (i, 0)),
            pl.BlockSpec((1, 1), lambda i: (0, 0)),
            pl.BlockSpec((1, 1), lambda i: (0, 0)),
        ],
        out_shape=[
            jax.ShapeDtypeStruct((ROWS, DIM), jnp.float32),
            jax.ShapeDtypeStruct((ROWS, N_CODES), jnp.float32),
            jax.ShapeDtypeStruct((ROWS, 1), jnp.int32),
            jax.ShapeDtypeStruct((1, 1), jnp.float32),
            jax.ShapeDtypeStruct((1, 1), jnp.float32),
        ],
        scratch_shapes=[
            pltpu.VMEM((1, N_CODES), jnp.float32),
            pltpu.SMEM((1,), jnp.float32),
        ],
        compiler_params=pltpu.CompilerParams(
            dimension_semantics=("arbitrary",)),
    )(zp_flat, codebook, row_nrm, code_nrm)

    z_q_out = jnp.transpose(zq_flat.reshape(bsz, h, w, ch), (0, 3, 1, 2))
    return (loss[0, 0], z_q_out, codes, idx, perp[0, 0])
